# baseline (speedup 1.0000x reference)
import functools

import jax
import jax.numpy as jnp
from jax import lax
from jax.experimental import pallas as pl
from jax.experimental.pallas import tpu as pltpu
from jax.experimental.pallas import tpu_sc as plsc

PMT = 16
D = 64


def _sc_gather(table_t, cell_idx):
    d, v = table_t.shape
    b = cell_idx.shape[0]
    info = plsc.get_sparse_core_info()
    nw = info.num_cores * info.num_subcores
    bpw = b // nw
    chunk = 8
    mesh = plsc.VectorSubcoreMesh(core_axis_name="c", subcore_axis_name="s")

    @functools.partial(
        pl.kernel,
        mesh=mesh,
        out_type=jax.ShapeDtypeStruct((d, b), jnp.float32),
        scratch_types=[
            pltpu.VMEM((bpw,), jnp.int32),
            pltpu.VMEM((d, chunk * 128), jnp.float32),
            pltpu.VMEM((d, bpw), jnp.float32),
            pltpu.SemaphoreType.DMA,
        ],
        compiler_params=pltpu.CompilerParams(use_tc_tiling_on_sc=True,
                                             needs_layout_passes=False),
    )
    def k(table_hbm, idx_hbm, out_hbm, idx_v, stage_v, tile_v, sem):
        wid = lax.axis_index("s") * info.num_cores + lax.axis_index("c")
        base = wid * bpw
        pltpu.sync_copy(idx_hbm.at[pl.ds(base, bpw)], idx_v)

        def body(j0, carry):
            vec = idx_v[pl.ds(j0 * 16, 16)]
            for half in range(2):
                for s in range(chunk):
                    t = half * chunk + s
                    col0 = pl.multiple_of((vec[t] // 128) * 128, 128)
                    pltpu.async_copy(
                        table_hbm.at[:, pl.ds(col0, 128)],
                        stage_v.at[:, pl.ds(s * 128, 128)],
                        sem,
                    )
                for s in range(chunk):
                    pltpu.make_async_copy(
                        table_hbm.at[:, pl.ds(0, 128)],
                        stage_v.at[:, pl.ds(s * 128, 128)],
                        sem,
                    ).wait()
                for s in range(chunk):
                    t = half * chunk + s
                    lane = vec[t] % 128
                    j = j0 * 16 + t
                    for g in range(d // 16):
                        rows = jnp.arange(16, dtype=jnp.int32) + g * 16
                        vals = plsc.load_gather(
                            stage_v, [rows, jnp.full((16,), s * 128 + lane,
                                                     jnp.int32)])
                        plsc.store_scatter(
                            tile_v, [rows, jnp.full((16,), j, jnp.int32)],
                            vals)
            return carry

        lax.fori_loop(0, bpw // 16, body, 0)
        pltpu.sync_copy(tile_v, out_hbm.at[:, pl.ds(base, bpw)])

    return k(table_t, cell_idx)


def _tc_src_copy(src_t):
    seq, d, b = src_t.shape
    tot = PMT + seq
    tb = 8
    grid = (seq // tb,)
    off = PMT // tb

    def body(src_ref, x_ref):
        x_ref[...] = src_ref[...]

    return pl.pallas_call(
        body,
        grid=grid,
        in_specs=[
            pl.BlockSpec((tb, d, b), lambda i: (i, 0, 0)),
        ],
        out_specs=pl.BlockSpec((tb, d, b), lambda i: (i + off, 0, 0)),
        out_shape=jax.ShapeDtypeStruct((tot, d, b), src_t.dtype),
    )(src_t)


def _tc_prompt_fill(x_part, cell_vec_t, prompt_base):
    tot, d, b = x_part.shape
    tb = 8
    grid = (PMT // tb,)

    def body(x_in, cvt_ref, pb_ref, x_ref):
        x_ref[...] = pb_ref[...][:, :, None] + cvt_ref[...][None, :, :]

    return pl.pallas_call(
        body,
        grid=grid,
        in_specs=[
            pl.BlockSpec(memory_space=pl.ANY),
            pl.BlockSpec((d, b), lambda i: (0, 0)),
            pl.BlockSpec((tb, d), lambda i: (i, 0)),
        ],
        out_specs=pl.BlockSpec((tb, d, b), lambda i: (i, 0, 0)),
        out_shape=jax.ShapeDtypeStruct((tot, d, b), x_part.dtype),
        input_output_aliases={0: 0},
    )(x_part, cell_vec_t, prompt_base)


def kernel(src_embs, src_mask, cell_idx, prompt_base, cell_embed_weight):
    table_t = cell_embed_weight.T
    src_t = jnp.transpose(src_embs, (1, 2, 0))
    mask_t = src_mask.T
    cell_vec_t = _sc_gather(table_t, cell_idx.astype(jnp.int32))
    x_p = _tc_src_copy(src_t)
    x_t = _tc_prompt_fill(x_p, cell_vec_t, prompt_base)
    x = jnp.transpose(x_t, (2, 0, 1))
    new_mask_t = jnp.concatenate(
        [jnp.ones((PMT, mask_t.shape[1]), mask_t.dtype), mask_t], axis=0)
    new_mask = new_mask_t.T
    return (x, new_mask)
